# speedup vs baseline: 1.5310x; 1.5310x over previous
import functools

import jax
import jax.numpy as jnp
from jax.experimental import pallas as pl
from jax.experimental.pallas import tpu as pltpu

E = 8
TOPK = 2
T = 2048
D = 1024
F = 1024
BT = 256


def _moe_body(ids_ref, tw_ref, x_ref, w1_ref, w2_ref, out_ref, acc_ref):
    e = pl.program_id(0)
    b = pl.program_id(1)

    x = x_ref[...]
    w1 = w1_ref[0]
    w2 = w2_ref[0]

    h = jax.lax.dot_general(x, w1, (((1,), (1,)), ((), ())),
                            preferred_element_type=jnp.float32)
    gate = h[:, :F]
    up = h[:, F:]
    act = gate * jax.nn.sigmoid(gate) * up
    y = jax.lax.dot_general(act, w2, (((1,), (1,)), ((), ())),
                            preferred_element_type=jnp.float32)

    coef = jnp.sum(jnp.where(ids_ref[...] == e, tw_ref[...], 0.0),
                   axis=-1, keepdims=True)
    contrib = coef * y

    sl = pl.ds(b * BT, BT)

    @pl.when(e == 0)
    def _():
        acc_ref[sl, :] = contrib

    @pl.when(e != 0)
    def _():
        acc_ref[sl, :] += contrib

    @pl.when(e == E - 1)
    def _():
        out_ref[...] = acc_ref[sl, :]


def kernel(hidden_states, topk_weights, topk_ids, w1, w2):
    topk_ids = topk_ids.astype(jnp.int32)
    nt = T // BT
    out = pl.pallas_call(
        _moe_body,
        grid=(E, nt),
        in_specs=[
            pl.BlockSpec((BT, TOPK), lambda e, b: (b, 0)),
            pl.BlockSpec((BT, TOPK), lambda e, b: (b, 0)),
            pl.BlockSpec((BT, D), lambda e, b: (b, 0)),
            pl.BlockSpec((1, 2 * F, D), lambda e, b: (e, 0, 0)),
            pl.BlockSpec((1, D, F), lambda e, b: (e, 0, 0)),
        ],
        out_specs=pl.BlockSpec((BT, D), lambda e, b: (b, 0)),
        out_shape=jax.ShapeDtypeStruct((T, D), jnp.float32),
        scratch_shapes=[pltpu.VMEM((T, D), jnp.float32)],
    )(topk_ids, topk_weights, hidden_states, w1, w2)
    return out
